# baseline (speedup 1.0000x reference)
import jax
import jax.numpy as jnp
from jax import lax
from jax.experimental import pallas as pl
from jax.experimental.pallas import tpu as pltpu
from jax.experimental.pallas import tpu_sc as plsc

NUM_CLAUSES_P1 = 9
HIDDEN = 2048
BATCH = 16384

_INFO = plsc.get_sparse_core_info()
NC = _INFO.num_cores
NS = _INFO.num_subcores
NW = NC * NS
B_PER_W = BATCH // NW
CHUNK = 24
NCHUNK = -(-B_PER_W // CHUNK)
NBUF = 2


def _sc_body(idx_hbm, table_hbm, out_hbm, idx_v,
             buf0, buf1, gs0, gs1, ss0, ss1):
    bufs = (buf0, buf1)
    gsems = (gs0, gs1)
    ssems = (ss0, ss1)
    cid = lax.axis_index("c")
    sid = lax.axis_index("s")
    wid = sid * NC + cid
    base = wid * B_PER_W

    pltpu.sync_copy(idx_hbm.at[pl.ds(base, B_PER_W)], idx_v)

    def chunk_start(c):
        return jnp.minimum(c * CHUNK, B_PER_W - CHUNK)

    def gather(c, b):
        return pltpu.make_async_copy(
            table_hbm.at[idx_v.at[pl.ds(chunk_start(c), CHUNK)]],
            bufs[b], gsems[b])

    def store(c, b):
        return pltpu.make_async_copy(
            bufs[b], out_hbm.at[pl.ds(base + chunk_start(c), CHUNK)],
            ssems[b])

    for b in range(NBUF):
        gather(b, b).start()

    def step(c, carry):
        b = lax.rem(c, NBUF)

        def run(b):
            gather(c, b).wait()
            st = store(c, b)
            st.start()

            @pl.when(c + NBUF < NCHUNK)
            def _():
                st.wait()
                gather(c + NBUF, b).start()

        for bb in range(NBUF):
            pl.when(b == bb)(lambda bb=bb: run(bb))
        return carry

    lax.fori_loop(0, NCHUNK, step, 0)

    for b in range(NBUF):
        store(NCHUNK - NBUF + b, b).wait()


@jax.jit
def kernel(clause_indices, embeddings):
    idx = clause_indices.astype(jnp.int32)
    mesh = plsc.VectorSubcoreMesh(core_axis_name="c", subcore_axis_name="s")
    f = pl.kernel(
        _sc_body,
        out_type=jax.ShapeDtypeStruct((BATCH, HIDDEN), jnp.float32),
        mesh=mesh,
        scratch_types=[
            pltpu.VMEM((B_PER_W,), jnp.int32),
            pltpu.VMEM((CHUNK, HIDDEN), jnp.float32),
            pltpu.VMEM((CHUNK, HIDDEN), jnp.float32),
            pltpu.SemaphoreType.DMA,
            pltpu.SemaphoreType.DMA,
            pltpu.SemaphoreType.DMA,
            pltpu.SemaphoreType.DMA,
        ],
    )
    return f(idx, embeddings)


# speedup vs baseline: 1.2250x; 1.2250x over previous
import jax
import jax.numpy as jnp
from jax import lax
from jax.experimental import pallas as pl
from jax.experimental.pallas import tpu as pltpu
from jax.experimental.pallas import tpu_sc as plsc

NUM_CLAUSES_P1 = 9
HIDDEN = 2048
LANES = 16
NGRP = HIDDEN // LANES
BATCH = 16384

_INFO = plsc.get_sparse_core_info()
NC = _INFO.num_cores
NS = _INFO.num_subcores
NW = NC * NS
B_PER_W = BATCH // NW
CHUNK = 16
NCHUNK = B_PER_W // CHUNK
NBUF = 2


def _sc_body(idx_hbm, table_hbm, out_hbm, table_v, idx_s,
             buf0, buf1, ss0, ss1):
    bufs = (buf0, buf1)
    ssems = (ss0, ss1)
    cid = lax.axis_index("c")
    sid = lax.axis_index("s")
    wid = sid * NC + cid
    base = wid * B_PER_W

    pltpu.sync_copy(table_hbm, table_v)
    pltpu.sync_copy(idx_hbm.at[pl.ds(base, B_PER_W)], idx_s)

    def fill(c, b):
        ivec = idx_s[pl.ds(c * CHUNK, LANES)]
        for r in range(CHUNK):
            i = ivec[r]

            def grp_step(g, carry2, r=r, i=i):
                bufs[b][r, pl.ds(g * LANES, LANES)] = (
                    table_v[i, pl.ds(g * LANES, LANES)])
                return carry2

            lax.fori_loop(0, NGRP, grp_step, 0, unroll=16)

    def store(c, b):
        return pltpu.make_async_copy(
            bufs[b], out_hbm.at[pl.ds(base + c * CHUNK, CHUNK)], ssems[b])

    for b in range(NBUF):
        fill(b, b)
        store(b, b).start()

    def step(c, carry):
        for bb in range(NBUF):
            @pl.when(lax.rem(c, NBUF) == bb)
            def _(bb=bb):
                store(c - NBUF, bb).wait()
                fill(c, bb)
                store(c, bb).start()
        return carry

    lax.fori_loop(NBUF, NCHUNK, step, 0)

    for b in range(NBUF):
        store(NCHUNK - NBUF + b, (NCHUNK - NBUF + b) % NBUF).wait()


@jax.jit
def kernel(clause_indices, embeddings):
    idx = clause_indices.astype(jnp.int32)
    mesh = plsc.VectorSubcoreMesh(core_axis_name="c", subcore_axis_name="s")
    f = pl.kernel(
        _sc_body,
        out_type=jax.ShapeDtypeStruct((BATCH, HIDDEN), jnp.float32),
        mesh=mesh,
        scratch_types=[
            pltpu.VMEM((NUM_CLAUSES_P1, HIDDEN), jnp.float32),
            pltpu.VMEM((B_PER_W,), jnp.int32),
            pltpu.VMEM((CHUNK, HIDDEN), jnp.float32),
            pltpu.VMEM((CHUNK, HIDDEN), jnp.float32),
            pltpu.SemaphoreType.DMA,
            pltpu.SemaphoreType.DMA,
        ],
    )
    return f(idx, embeddings)
